# Initial kernel scaffold; baseline (speedup 1.0000x reference)
import jax
import jax.numpy as jnp
from jax.experimental import pallas as pl


def kernel(x, edge_index, W1l, b1l, W1r, b1r, gamma, beta, W2l, b2l, W2r, b2r):
    raise NotImplementedError("write your pallas kernel here")


# speedup vs baseline: 6.8916x; 6.8916x over previous
import functools
import jax
import jax.numpy as jnp
from jax import lax
from jax.experimental import pallas as pl
from jax.experimental.pallas import tpu as pltpu
from jax.experimental.pallas import tpu_sc as plsc

N_CORES = 2
N_SUBCORES = 16
N_TILES = N_CORES * N_SUBCORES
CHUNK = 125


def _make_seg_sum(n_acc: int, d: int, chunks_per_tile: int):
  rows_per_sub = n_acc // N_SUBCORES
  mesh = plsc.VectorSubcoreMesh(core_axis_name="c", subcore_axis_name="s")

  def body(rows_hbm, src_hbm, dst_hbm, zeros_hbm, out_hbm,
           src_v, dst_v, buf_v, acc_sh, gsem):
    cid = lax.axis_index("c")
    sid = lax.axis_index("s")
    wid = cid * N_SUBCORES + sid
    pltpu.sync_copy(zeros_hbm.at[pl.ds(sid * rows_per_sub, rows_per_sub)],
                    acc_sh.at[pl.ds(sid * rows_per_sub, rows_per_sub)])
    pltpu.sync_copy(src_hbm.at[pl.ds(wid * chunks_per_tile, chunks_per_tile)],
                    src_v)
    pltpu.sync_copy(dst_hbm.at[pl.ds(wid * chunks_per_tile, chunks_per_tile)],
                    dst_v)
    plsc.subcore_barrier()

    @pl.loop(0, chunks_per_tile)
    def _chunk(c):
      pltpu.async_copy(rows_hbm.at[src_v.at[c]], buf_v, gsem).wait()
      pltpu.sync_copy(buf_v, acc_sh.at[dst_v.at[c]], add=True)

    plsc.subcore_barrier()
    pltpu.sync_copy(acc_sh.at[pl.ds(sid * rows_per_sub, rows_per_sub)],
                    out_hbm.at[cid, pl.ds(sid * rows_per_sub, rows_per_sub)])

  return pl.kernel(
      body,
      out_type=jax.ShapeDtypeStruct((N_CORES, n_acc, d), jnp.float32),
      mesh=mesh,
      compiler_params=pltpu.CompilerParams(use_tc_tiling_on_sc=False),
      scratch_types=[
          pltpu.VMEM((chunks_per_tile, CHUNK), jnp.int32),
          pltpu.VMEM((chunks_per_tile, CHUNK), jnp.int32),
          pltpu.VMEM((CHUNK, d), jnp.float32),
          pltpu.VMEM_SHARED((n_acc, d), jnp.float32),
          pltpu.SemaphoreType.DMA,
      ],
  )


_DOT = functools.partial(jnp.dot, preferred_element_type=jnp.float32,
                         precision=lax.Precision.HIGHEST)


def _l1_body(x_ref, wl_ref, wr_ref, b_ref, pext_ref, base_ref):
  x = x_ref[...]
  p = _DOT(x, wl_ref[...])
  ones = jnp.ones((x.shape[0], 16), jnp.float32)
  pext_ref[...] = jnp.concatenate([p, ones], axis=1)
  base_ref[...] = _DOT(x, wr_ref[...]) + b_ref[...]


def _mid_body(p0_ref, p1_ref, base_ref, g_ref, bt_ref, wl_ref, wr_ref, b2_ref,
              q_ref, s_ref, cnt_ref):
  hid = base_ref.shape[1]
  agg = p0_ref[:, :hid] + p1_ref[:, :hid]
  cnt = jnp.maximum(p0_ref[:, hid:hid + 1] + p1_ref[:, hid:hid + 1], 1.0)
  h = agg / cnt + base_ref[...]
  mu = jnp.mean(h, axis=0, keepdims=True)
  var = jnp.mean((h - mu) ** 2, axis=0, keepdims=True)
  h = (h - mu) / jnp.sqrt(var + 1e-5) * g_ref[...] + bt_ref[...]
  h = jnp.maximum(h, 0.0)
  q_ref[...] = _DOT(h, wl_ref[...])
  s_ref[...] = _DOT(h, wr_ref[...]) + b2_ref[...]
  cnt_ref[...] = cnt


def _out_body(q0_ref, q1_ref, s_ref, cnt_ref, out_ref):
  z = (q0_ref[...] + q1_ref[...]) / cnt_ref[...] + s_ref[...]
  norm = jnp.sqrt(jnp.sum(z * z, axis=1, keepdims=True))
  out_ref[...] = z / jnp.maximum(norm, 1e-12)


@jax.jit
def kernel(x, edge_index, W1l, b1l, W1r, b1r, gamma, beta, W2l, b2l, W2r, b2r):
  n, in_dim = x.shape
  hid = W1l.shape[1]
  out_dim = W2l.shape[1]
  n_edges = edge_index.shape[1]

  e_pad = -(-n_edges // (N_TILES * CHUNK)) * (N_TILES * CHUNK)
  n_acc = -(-(n + (1 if e_pad != n_edges else 0)) //
            (8 * N_SUBCORES)) * (8 * N_SUBCORES)

  src = edge_index[0].astype(jnp.int32)
  dst = edge_index[1].astype(jnp.int32)
  if e_pad != n_edges:
    src = jnp.concatenate([src, jnp.zeros((e_pad - n_edges,), jnp.int32)])
    dst = jnp.concatenate(
        [dst, jnp.full((e_pad - n_edges,), n_acc - 1, jnp.int32)])
  total_chunks = e_pad // CHUNK
  chunks_per_tile = total_chunks // N_TILES
  src2d = src.reshape(total_chunks, CHUNK)
  dst2d = dst.reshape(total_chunks, CHUNK)

  dp1 = hid + 16
  zeros1 = jnp.zeros((n_acc, dp1), jnp.float32)
  zeros2 = jnp.zeros((n_acc, out_dim), jnp.float32)

  blk = 1000
  grid = n // blk
  b1 = (b1l + b1r).reshape(1, hid)
  pext_pad, base = pl.pallas_call(
      _l1_body,
      grid=(grid,),
      in_specs=[
          pl.BlockSpec((blk, in_dim), lambda i: (i, 0)),
          pl.BlockSpec((in_dim, hid), lambda i: (0, 0)),
          pl.BlockSpec((in_dim, hid), lambda i: (0, 0)),
          pl.BlockSpec((1, hid), lambda i: (0, 0)),
      ],
      out_specs=[
          pl.BlockSpec((blk, dp1), lambda i: (i, 0)),
          pl.BlockSpec((blk, hid), lambda i: (i, 0)),
      ],
      out_shape=[
          jax.ShapeDtypeStruct((n, dp1), jnp.float32),
          jax.ShapeDtypeStruct((n, hid), jnp.float32),
      ],
  )(x, W1l, W1r, b1)
  pext = jnp.zeros((n_acc, dp1), jnp.float32).at[:n].set(pext_pad)

  part1 = _make_seg_sum(n_acc, dp1, chunks_per_tile)(
      pext, src2d, dst2d, zeros1)

  b2 = (b2l + b2r).reshape(1, out_dim)
  q, s, cnt = pl.pallas_call(
      _mid_body,
      compiler_params=pltpu.CompilerParams(
          vmem_limit_bytes=100 * 1024 * 1024),
      in_specs=[pl.BlockSpec((n, dp1), lambda: (0, 0))] * 2 + [
          pl.BlockSpec((n, hid), lambda: (0, 0)),
          pl.BlockSpec((1, hid), lambda: (0, 0)),
          pl.BlockSpec((1, hid), lambda: (0, 0)),
          pl.BlockSpec((hid, out_dim), lambda: (0, 0)),
          pl.BlockSpec((hid, out_dim), lambda: (0, 0)),
          pl.BlockSpec((1, out_dim), lambda: (0, 0)),
      ],
      out_specs=[
          pl.BlockSpec((n, out_dim), lambda: (0, 0)),
          pl.BlockSpec((n, out_dim), lambda: (0, 0)),
          pl.BlockSpec((n, 1), lambda: (0, 0)),
      ],
      out_shape=[
          jax.ShapeDtypeStruct((n, out_dim), jnp.float32),
          jax.ShapeDtypeStruct((n, out_dim), jnp.float32),
          jax.ShapeDtypeStruct((n, 1), jnp.float32),
      ],
  )(part1[0, :n], part1[1, :n], base, gamma.reshape(1, hid),
    beta.reshape(1, hid), W2l, W2r, b2)

  qp = jnp.zeros((n_acc, out_dim), jnp.float32).at[:n].set(q)

  part2 = _make_seg_sum(n_acc, out_dim, chunks_per_tile)(
      qp, src2d, dst2d, zeros2)

  z = pl.pallas_call(
      _out_body,
      grid=(grid,),
      in_specs=[
          pl.BlockSpec((blk, out_dim), lambda i: (i, 0)),
          pl.BlockSpec((blk, out_dim), lambda i: (i, 0)),
          pl.BlockSpec((blk, out_dim), lambda i: (i, 0)),
          pl.BlockSpec((blk, 1), lambda i: (i, 0)),
      ],
      out_specs=pl.BlockSpec((blk, out_dim), lambda i: (i, 0)),
      out_shape=jax.ShapeDtypeStruct((n, out_dim), jnp.float32),
  )(part2[0, :n], part2[1, :n], s, cnt)
  return z
